# baseline (speedup 1.0000x reference)
import functools

import jax
import jax.numpy as jnp
from jax import lax
from jax.experimental import pallas as pl
from jax.experimental.pallas import tpu as pltpu

NUM_CODES = 8192
CODE_DIM = 256
BETA = 0.25
EPS = 1e-6

TM = 512
TN = 2048
TN2 = 2048


def _norm_rows(x, eps=EPS):
    n = jnp.sqrt(jnp.sum(x * x, axis=1, keepdims=True))
    return x / jnp.maximum(n, eps)


def _argmin_body(h_ref, cb_ref, idx_ref, cbn_ref, bval_ref, bidx_ref):
    j = pl.program_id(0)
    i = pl.program_id(1)
    nj = pl.num_programs(0)

    h = h_ref[...]
    hn = _norm_rows(h)
    h_sq = jnp.sum(hn * hn, axis=1, keepdims=True)

    cb = cb_ref[...]
    cbn = _norm_rows(cb)
    c_sq = jnp.sum(cbn * cbn, axis=1)

    @pl.when(i == 0)
    def _():
        cbn_ref[...] = cbn

    dot = lax.dot_general(hn, cbn, (((1,), (1,)), ((), ())),
                          preferred_element_type=jnp.float32)
    dist = (h_sq + c_sq[None, :]) - 2.0 * dot

    local_min = jnp.min(dist, axis=1)
    cols = lax.broadcasted_iota(jnp.int32, (TM, TN), 1)
    local_idx = jnp.min(
        jnp.where(dist == local_min[:, None], cols, TN), axis=1) + j * TN

    sl = pl.ds(i * TM, TM)
    prev_val = bval_ref[sl]
    prev_idx = bidx_ref[sl]
    take = jnp.logical_or(j == 0, local_min < prev_val)
    best_val = jnp.where(take, local_min, prev_val)
    best_idx = jnp.where(take, local_idx, prev_idx)
    bval_ref[sl] = best_val
    bidx_ref[sl] = best_idx
    idx_ref[0, 0, :] = best_idx


def _emit_body(idx_ref, cb_ref, h_ref, q_ref, ch_ref, loss_ref, acc_ref):
    j = pl.program_id(1)
    nj = pl.num_programs(1)

    idx_v = idx_ref[0, 0, :]
    cols = lax.broadcasted_iota(jnp.int32, (TM, TN2), 1) + j * TN2
    q = (idx_v[:, None] == cols).astype(jnp.float32)
    q_ref[...] = q

    cbn = _norm_rows(cb_ref[...])
    part = lax.dot_general(q, cbn, (((1,), (0,)), ((), ())),
                           preferred_element_type=jnp.float32)

    @pl.when(j == 0)
    def _():
        acc_ref[...] = part

    @pl.when(j > 0)
    def _():
        acc_ref[...] += part

    @pl.when(j == nj - 1)
    def _():
        ch = acc_ref[...]
        ch_ref[...] = ch
        hn = _norm_rows(h_ref[...])
        d = hn - ch
        loss_ref[0, 0, :] = jnp.full((128,), jnp.sum(d * d), jnp.float32)


def kernel(h, codebook):
    B0, B1, D = h.shape
    T = B0 * B1
    h_flat = h.reshape(T, D)
    ni = T // TM
    nj = NUM_CODES // TN
    nj2 = NUM_CODES // TN2

    idx3, cbn = pl.pallas_call(
        _argmin_body,
        grid=(nj, ni),
        in_specs=[
            pl.BlockSpec((TM, D), lambda j, i: (i, 0)),
            pl.BlockSpec((TN, D), lambda j, i: (j, 0)),
        ],
        out_specs=[
            pl.BlockSpec((1, 1, TM), lambda j, i: (i, 0, 0)),
            pl.BlockSpec((TN, D), lambda j, i: (j, 0)),
        ],
        out_shape=[
            jax.ShapeDtypeStruct((ni, 1, TM), jnp.int32),
            jax.ShapeDtypeStruct((NUM_CODES, D), jnp.float32),
        ],
        scratch_shapes=[
            pltpu.VMEM((T,), jnp.float32),
            pltpu.VMEM((T,), jnp.int32),
        ],
    )(h_flat, codebook)

    q2, c_hard, loss_p = pl.pallas_call(
        _emit_body,
        grid=(ni, nj2),
        in_specs=[
            pl.BlockSpec((1, 1, TM), lambda i, j: (i, 0, 0)),
            pl.BlockSpec((TN2, D), lambda i, j: (j, 0)),
            pl.BlockSpec((TM, D), lambda i, j: (i, 0)),
        ],
        out_specs=[
            pl.BlockSpec((TM, TN2), lambda i, j: (i, j)),
            pl.BlockSpec((TM, D), lambda i, j: (i, 0)),
            pl.BlockSpec((1, 1, 128), lambda i, j: (i, 0, 0)),
        ],
        out_shape=[
            jax.ShapeDtypeStruct((T, NUM_CODES), jnp.float32),
            jax.ShapeDtypeStruct((T, D), jnp.float32),
            jax.ShapeDtypeStruct((ni, 1, 128), jnp.float32),
        ],
        scratch_shapes=[
            pltpu.VMEM((TM, D), jnp.float32),
        ],
    )(idx3, cbn, h_flat)

    q = q2.reshape(B0, B1, NUM_CODES)
    c3 = c_hard.reshape(B0, B1, D)
    m = jnp.sum(loss_p[:, 0, 0]) / jnp.float32(T * D)
    loss = m + BETA * m
    indices_flat = idx3.reshape(T)
    return (q, c3, c3, c3, loss, indices_flat)


# speedup vs baseline: 1.0150x; 1.0150x over previous
import functools

import jax
import jax.numpy as jnp
from jax import lax
from jax.experimental import pallas as pl
from jax.experimental.pallas import tpu as pltpu

NUM_CODES = 8192
CODE_DIM = 256
BETA = 0.25
EPS = 1e-6

TM = 512
TN = 2048
TN2 = 2048


def _norm_rows(x, eps=EPS):
    n = jnp.sqrt(jnp.sum(x * x, axis=1, keepdims=True))
    return x / jnp.maximum(n, eps)


def _argmin_body(h_ref, cb_ref, idx_ref, cbn_ref, bval_ref, bidx_ref):
    j = pl.program_id(0)
    i = pl.program_id(1)
    nj = pl.num_programs(0)

    h = h_ref[...]
    hn = _norm_rows(h)
    h_sq = jnp.sum(hn * hn, axis=1, keepdims=True)

    cb = cb_ref[...]
    cbn = _norm_rows(cb)
    c_sq = jnp.sum(cbn * cbn, axis=1)

    @pl.when(i == 0)
    def _():
        cbn_ref[...] = cbn

    dot = lax.dot_general(hn, cbn, (((1,), (1,)), ((), ())),
                          preferred_element_type=jnp.float32)
    dist = (h_sq + c_sq[None, :]) - 2.0 * dot

    local_min = jnp.min(dist, axis=1)
    cols = lax.broadcasted_iota(jnp.int32, (TM, TN), 1)
    local_idx = jnp.min(
        jnp.where(dist == local_min[:, None], cols, TN), axis=1) + j * TN

    sl = pl.ds(i * TM, TM)
    prev_val = bval_ref[sl]
    prev_idx = bidx_ref[sl]
    take = jnp.logical_or(j == 0, local_min < prev_val)
    best_val = jnp.where(take, local_min, prev_val)
    best_idx = jnp.where(take, local_idx, prev_idx)
    bval_ref[sl] = best_val
    bidx_ref[sl] = best_idx
    idx_ref[0, 0, :] = best_idx


def _emit_body(idx_ref, cb_ref, h_ref, q_ref, ch_ref, loss_ref, acc_ref):
    j = pl.program_id(1)
    nj = pl.num_programs(1)

    idx_v = idx_ref[0, 0, :]
    cols = lax.broadcasted_iota(jnp.int32, (TM, TN2), 1) + j * TN2
    q = (idx_v[:, None] == cols).astype(jnp.float32)
    q_ref[...] = q

    cbn = cb_ref[...]
    part = lax.dot_general(q, cbn, (((1,), (0,)), ((), ())),
                           preferred_element_type=jnp.float32)

    @pl.when(j == 0)
    def _():
        acc_ref[...] = part

    @pl.when(j > 0)
    def _():
        acc_ref[...] += part

    @pl.when(j == nj - 1)
    def _():
        ch = acc_ref[...]
        ch_ref[...] = ch
        hn = _norm_rows(h_ref[...])
        d = hn - ch
        loss_ref[0, 0, :] = jnp.full((128,), jnp.sum(d * d), jnp.float32)


def kernel(h, codebook):
    B0, B1, D = h.shape
    T = B0 * B1
    h_flat = h.reshape(T, D)
    ni = T // TM
    nj = NUM_CODES // TN
    nj2 = NUM_CODES // TN2

    idx3, cbn = pl.pallas_call(
        _argmin_body,
        grid=(nj, ni),
        in_specs=[
            pl.BlockSpec((TM, D), lambda j, i: (i, 0)),
            pl.BlockSpec((TN, D), lambda j, i: (j, 0)),
        ],
        out_specs=[
            pl.BlockSpec((1, 1, TM), lambda j, i: (i, 0, 0)),
            pl.BlockSpec((TN, D), lambda j, i: (j, 0)),
        ],
        out_shape=[
            jax.ShapeDtypeStruct((ni, 1, TM), jnp.int32),
            jax.ShapeDtypeStruct((NUM_CODES, D), jnp.float32),
        ],
        scratch_shapes=[
            pltpu.VMEM((T,), jnp.float32),
            pltpu.VMEM((T,), jnp.int32),
        ],
    )(h_flat, codebook)

    q2, c_hard, loss_p = pl.pallas_call(
        _emit_body,
        grid=(ni, nj2),
        in_specs=[
            pl.BlockSpec((1, 1, TM), lambda i, j: (i, 0, 0)),
            pl.BlockSpec((TN2, D), lambda i, j: (j, 0)),
            pl.BlockSpec((TM, D), lambda i, j: (i, 0)),
        ],
        out_specs=[
            pl.BlockSpec((TM, TN2), lambda i, j: (i, j)),
            pl.BlockSpec((TM, D), lambda i, j: (i, 0)),
            pl.BlockSpec((1, 1, 128), lambda i, j: (i, 0, 0)),
        ],
        out_shape=[
            jax.ShapeDtypeStruct((T, NUM_CODES), jnp.float32),
            jax.ShapeDtypeStruct((T, D), jnp.float32),
            jax.ShapeDtypeStruct((ni, 1, 128), jnp.float32),
        ],
        scratch_shapes=[
            pltpu.VMEM((TM, D), jnp.float32),
        ],
    )(idx3, cbn, h_flat)

    q = q2.reshape(B0, B1, NUM_CODES)
    c3 = c_hard.reshape(B0, B1, D)
    m = jnp.sum(loss_p[:, 0, 0]) / jnp.float32(T * D)
    loss = m + BETA * m
    indices_flat = idx3.reshape(T)
    return (q, c3, c3, c3, loss, indices_flat)


# speedup vs baseline: 1.2655x; 1.2468x over previous
import jax
import jax.numpy as jnp
from jax import lax
from jax.experimental import pallas as pl
from jax.experimental.pallas import tpu as pltpu

NUM_CODES = 8192
CODE_DIM = 256
BETA = 0.25
EPS = 1e-6

TM = 512
TN = 2048
TN2 = 2048


def _norm_rows(x, eps=EPS):
    n = jnp.sqrt(jnp.sum(x * x, axis=1, keepdims=True))
    return x / jnp.maximum(n, eps)


def _argmin_body(h_ref, cb_ref, idx_ref, cbn_ref, hn_ref,
                 hsq_ref, csq_ref, bval_ref, bidx_ref):
    i = pl.program_id(0)
    j = pl.program_id(1)
    nj = pl.num_programs(1)

    @pl.when(i == 0)
    def _():
        cb = cb_ref[pl.ds(j * TN, TN), :]
        cbn = _norm_rows(cb)
        cbn_ref[pl.ds(j * TN, TN), :] = cbn
        csq_ref[pl.ds(j * TN, TN)] = jnp.sum(cbn * cbn, axis=1)

    @pl.when(j == 0)
    def _():
        hn = _norm_rows(h_ref[...])
        hn_ref[...] = hn
        hsq_ref[...] = jnp.sum(hn * hn, axis=1)

    hn = hn_ref[...]
    h_sq = hsq_ref[...][:, None]
    cbn = cbn_ref[pl.ds(j * TN, TN), :]
    c_sq = csq_ref[pl.ds(j * TN, TN)]

    dot = lax.dot_general(hn, cbn, (((1,), (1,)), ((), ())),
                          preferred_element_type=jnp.float32)
    dist = (h_sq + c_sq[None, :]) - 2.0 * dot

    local_idx = jnp.argmin(dist, axis=1).astype(jnp.int32) + j * TN
    local_min = jnp.min(dist, axis=1)

    prev_val = bval_ref[...]
    prev_idx = bidx_ref[...]
    take = jnp.logical_or(j == 0, local_min < prev_val)
    bval_ref[...] = jnp.where(take, local_min, prev_val)
    bidx_ref[...] = jnp.where(take, local_idx, prev_idx)

    @pl.when(j == nj - 1)
    def _():
        idx_ref[0, 0, :] = bidx_ref[...]


def _emit_body(idx_ref, cbn_ref, hn_ref, q_ref, ch_ref, loss_ref, acc_ref):
    j = pl.program_id(1)
    nj = pl.num_programs(1)

    idx_v = idx_ref[0, 0, :]
    cols = lax.broadcasted_iota(jnp.int32, (TM, TN2), 1) + j * TN2
    q = (idx_v[:, None] == cols).astype(jnp.float32)
    q_ref[...] = q

    cbn = cbn_ref[pl.ds(j * TN2, TN2), :]
    part = lax.dot_general(q, cbn, (((1,), (0,)), ((), ())),
                           preferred_element_type=jnp.float32)

    @pl.when(j == 0)
    def _():
        acc_ref[...] = part

    @pl.when(j > 0)
    def _():
        acc_ref[...] += part

    @pl.when(j == nj - 1)
    def _():
        ch = acc_ref[...]
        ch_ref[...] = ch
        d = hn_ref[...] - ch
        per_tok = jnp.sum(d * d, axis=1)
        loss_ref[0, 0, :] = jnp.sum(per_tok.reshape(-1, 128), axis=0)


def kernel(h, codebook):
    B0, B1, D = h.shape
    T = B0 * B1
    h_flat = h.reshape(T, D)
    ni = T // TM
    nj = NUM_CODES // TN
    nj2 = NUM_CODES // TN2

    idx3, cbn, hn = pl.pallas_call(
        _argmin_body,
        grid=(ni, nj),
        in_specs=[
            pl.BlockSpec((TM, D), lambda i, j: (i, 0)),
            pl.BlockSpec((NUM_CODES, D), lambda i, j: (0, 0)),
        ],
        out_specs=[
            pl.BlockSpec((1, 1, TM), lambda i, j: (i, 0, 0)),
            pl.BlockSpec((NUM_CODES, D), lambda i, j: (0, 0)),
            pl.BlockSpec((TM, D), lambda i, j: (i, 0)),
        ],
        out_shape=[
            jax.ShapeDtypeStruct((ni, 1, TM), jnp.int32),
            jax.ShapeDtypeStruct((NUM_CODES, D), jnp.float32),
            jax.ShapeDtypeStruct((T, D), jnp.float32),
        ],
        scratch_shapes=[
            pltpu.VMEM((TM,), jnp.float32),
            pltpu.VMEM((NUM_CODES,), jnp.float32),
            pltpu.VMEM((TM,), jnp.float32),
            pltpu.VMEM((TM,), jnp.int32),
        ],
    )(h_flat, codebook)

    q2, c_hard, loss_p = pl.pallas_call(
        _emit_body,
        grid=(ni, nj2),
        in_specs=[
            pl.BlockSpec((1, 1, TM), lambda i, j: (i, 0, 0)),
            pl.BlockSpec((NUM_CODES, D), lambda i, j: (0, 0)),
            pl.BlockSpec((TM, D), lambda i, j: (i, 0)),
        ],
        out_specs=[
            pl.BlockSpec((TM, TN2), lambda i, j: (i, j)),
            pl.BlockSpec((TM, D), lambda i, j: (i, 0)),
            pl.BlockSpec((1, 1, 128), lambda i, j: (i, 0, 0)),
        ],
        out_shape=[
            jax.ShapeDtypeStruct((T, NUM_CODES), jnp.float32),
            jax.ShapeDtypeStruct((T, D), jnp.float32),
            jax.ShapeDtypeStruct((ni, 1, 128), jnp.float32),
        ],
        scratch_shapes=[
            pltpu.VMEM((TM, D), jnp.float32),
        ],
    )(idx3, cbn, hn)

    q = q2.reshape(B0, B1, NUM_CODES)
    c3 = c_hard.reshape(B0, B1, D)
    m = jnp.sum(loss_p[:, 0, :]) / jnp.float32(T * D)
    loss = m + BETA * m
    indices_flat = idx3.reshape(T)
    return (q, c3, c3, c3, loss, indices_flat)
